# baseline (speedup 1.0000x reference)
import functools

import jax
import jax.numpy as jnp
from jax import lax
from jax.experimental import pallas as pl
from jax.experimental.pallas import tpu as pltpu
from jax.experimental.pallas import tpu_sc as plsc

_B = 128
_N = 32768
_L = 16
_NVEC = _N // _L
_NW = 32
_SC_ROWS_PER_W = 3
_SC_ROWS = _SC_ROWS_PER_W * _NW
_BISECT_ITERS = 22
_UNROLL = 8

_TC_BLOCK = 8
_TC_ITERS = 16


def _row_sparsemax(row_v, cand_v):
    @plsc.parallel_loop(
        0, _NVEC, unroll=_UNROLL,
        carry=(jnp.full((_L,), -3e38, jnp.float32), jnp.int32(0)),
    )
    def filt(i, carry):
        m_run, w = carry
        v = row_v[pl.ds(i * _L, _L)]
        m_run = jnp.maximum(m_run, v)
        mask = v >= (m_run - 1.0)
        plsc.store_compressed(cand_v.at[pl.ds(w, _L)], v, mask=mask)
        pc = plsc.all_reduce_population_count(mask)
        return m_run, w + pc[0]

    m_run, w = filt
    m = jnp.max(m_run)
    iota = lax.iota(jnp.int32, _L)
    plsc.store_scatter(cand_v, [w + iota], jnp.full((_L,), m - 2.0))
    nvec = w // _L + 1

    def bisect(_, carry):
        lo, hi = carry
        t = 0.5 * (lo + hi)

        def acc_fn(j, acc):
            v = cand_v[pl.ds(j * _L, _L)]
            return acc + jnp.maximum(v - t, 0.0)

        acc = lax.fori_loop(0, nvec, acc_fn, jnp.zeros((_L,), jnp.float32))
        s = jnp.sum(acc)
        ge = s >= 1.0
        return jnp.where(ge, t, lo), jnp.where(ge, hi, t)

    lo, _ = lax.fori_loop(0, _BISECT_ITERS, bisect, (m - 1.0, m))

    def ks_fn(j, carry):
        ak, asum = carry
        v = cand_v[pl.ds(j * _L, _L)]
        above = v > lo
        ak = ak + above.astype(jnp.float32)
        asum = asum + jnp.where(above, v, 0.0)
        return ak, asum

    ak, asum = lax.fori_loop(
        0, nvec, ks_fn,
        (jnp.zeros((_L,), jnp.float32), jnp.zeros((_L,), jnp.float32)),
    )
    tau = (jnp.full((_L,), jnp.sum(asum)) - 1.0) / jnp.full((_L,), jnp.sum(ak))

    @plsc.parallel_loop(0, _NVEC, unroll=_UNROLL)
    def out_loop(i):
        v = row_v[pl.ds(i * _L, _L)]
        row_v[pl.ds(i * _L, _L)] = jnp.maximum(v - tau, 0.0)


def _sc_part(x):
    mesh = plsc.VectorSubcoreMesh(core_axis_name="c", subcore_axis_name="s")

    @functools.partial(
        pl.kernel,
        mesh=mesh,
        out_type=jax.ShapeDtypeStruct((_SC_ROWS, _N), jnp.float32),
        scratch_types=[
            pltpu.VMEM((_N,), jnp.float32),
            pltpu.VMEM((_N,), jnp.float32),
            pltpu.VMEM((_N + _L,), jnp.float32),
            pltpu.SemaphoreType.DMA,
            pltpu.SemaphoreType.DMA,
            pltpu.SemaphoreType.DMA,
            pltpu.SemaphoreType.DMA,
        ],
        compiler_params=pltpu.CompilerParams(needs_layout_passes=False),
    )
    def run(x_hbm, out_hbm, row_a, row_b, cand_v, si_a, si_b, so_a, so_b):
        wid = lax.axis_index("s") * 2 + lax.axis_index("c")
        base = wid * _SC_ROWS_PER_W
        bufs = (row_a, row_b)
        sin = (si_a, si_b)
        sout = (so_a, so_b)

        def cp_in(r, b):
            return pltpu.make_async_copy(x_hbm.at[base + r], bufs[b], sin[b])

        def cp_out(r, b):
            return pltpu.make_async_copy(bufs[b], out_hbm.at[base + r], sout[b])

        cp_in(0, 0).start()
        cp_in(1, 1).start()
        for r in range(_SC_ROWS_PER_W):
            b = r % 2
            cp_in(r, b).wait()
            if r >= 1 and r + 1 < _SC_ROWS_PER_W:
                cp_out(r - 1, 1 - b).wait()
                cp_in(r + 1, 1 - b).start()
            _row_sparsemax(bufs[b], cand_v)
            cp_out(r, b).start()
        cp_out(_SC_ROWS_PER_W - 2, _SC_ROWS_PER_W % 2).wait()
        cp_out(_SC_ROWS_PER_W - 1, 1 - _SC_ROWS_PER_W % 2).wait()

    return run(x)


def _tc_rows(x_ref, o_ref):
    x = x_ref[...]
    m = jnp.max(x, axis=1, keepdims=True)

    def body(_, carry):
        lo, hi = carry
        t = 0.5 * (lo + hi)
        s = jnp.sum(jnp.maximum(x - t, 0.0), axis=1, keepdims=True)
        ge = s >= 1.0
        return jnp.where(ge, t, lo), jnp.where(ge, hi, t)

    lo, _ = lax.fori_loop(0, _TC_ITERS, body, (m - 1.0, m))
    above = x > lo
    k = jnp.sum(above.astype(jnp.float32), axis=1, keepdims=True)
    s = jnp.sum(jnp.where(above, x, 0.0), axis=1, keepdims=True)
    tau = (s - 1.0) / k
    o_ref[...] = jnp.maximum(x - tau, 0.0)


def _tc_part(x):
    b, n = x.shape
    return pl.pallas_call(
        _tc_rows,
        grid=(b // _TC_BLOCK,),
        in_specs=[pl.BlockSpec((_TC_BLOCK, n), lambda i: (i, 0))],
        out_specs=pl.BlockSpec((_TC_BLOCK, n), lambda i: (i, 0)),
        out_shape=jax.ShapeDtypeStruct((b, n), x.dtype),
    )(x)


def kernel(inputs):
    out_sc = _sc_part(inputs[:_SC_ROWS])
    out_tc = _tc_part(inputs[_SC_ROWS:])
    return jnp.concatenate([out_sc, out_tc], axis=0)


# speedup vs baseline: 1.6634x; 1.6634x over previous
import functools

import jax
import jax.numpy as jnp
from jax import lax
from jax.experimental import pallas as pl
from jax.experimental.pallas import tpu as pltpu
from jax.experimental.pallas import tpu_sc as plsc

_B = 128
_N = 32768
_L = 16
_NVEC = _N // _L
_NW = 32
_ROWS_PER_W = _B // _NW
_BISECT_ITERS = 16
_UNROLL = 8


def _row_sparsemax(row_v, cand_v):
    @plsc.parallel_loop(
        0, _NVEC, unroll=_UNROLL, carry=jnp.full((_L,), -3e38, jnp.float32)
    )
    def maxp(i, m_run):
        return jnp.maximum(m_run, row_v[pl.ds(i * _L, _L)])

    m = jnp.max(maxp)
    thr = jnp.full((_L,), m - 1.0)

    @plsc.parallel_loop(
        0, _NVEC, unroll=_UNROLL, carry=jnp.zeros((_L,), jnp.int32)
    )
    def filt(i, w):
        v = row_v[pl.ds(i * _L, _L)]
        mask = v > thr
        plsc.store_compressed(cand_v.at[pl.ds(w[0], _L)], v, mask=mask)
        return w + plsc.all_reduce_population_count(mask)

    w = filt[0]
    iota = lax.iota(jnp.int32, _L)
    plsc.store_scatter(cand_v, [w + iota], jnp.full((_L,), m - 2.0))
    nvec = w // _L + 1

    def bisect(_, carry):
        lo, hi = carry
        t = 0.5 * (lo + hi)

        def acc_fn(j, acc):
            v = cand_v[pl.ds(j * _L, _L)]
            return acc + jnp.maximum(v - t, 0.0)

        acc = lax.fori_loop(0, nvec, acc_fn, jnp.zeros((_L,), jnp.float32))
        s = jnp.sum(acc)
        ge = s >= 1.0
        return jnp.where(ge, t, lo), jnp.where(ge, hi, t)

    lo, _ = lax.fori_loop(0, _BISECT_ITERS, bisect, (m - 1.0, m))

    def ks_fn(j, carry):
        ak, asum = carry
        v = cand_v[pl.ds(j * _L, _L)]
        above = v > lo
        ak = ak + above.astype(jnp.float32)
        asum = asum + jnp.where(above, v, 0.0)
        return ak, asum

    ak, asum = lax.fori_loop(
        0, nvec, ks_fn,
        (jnp.zeros((_L,), jnp.float32), jnp.zeros((_L,), jnp.float32)),
    )
    tau = (jnp.full((_L,), jnp.sum(asum)) - 1.0) / jnp.full((_L,), jnp.sum(ak))

    @plsc.parallel_loop(0, _NVEC, unroll=_UNROLL)
    def out_loop(i):
        v = row_v[pl.ds(i * _L, _L)]
        row_v[pl.ds(i * _L, _L)] = jnp.maximum(v - tau, 0.0)


def kernel(inputs):
    mesh = plsc.VectorSubcoreMesh(core_axis_name="c", subcore_axis_name="s")

    @functools.partial(
        pl.kernel,
        mesh=mesh,
        out_type=jax.ShapeDtypeStruct((_B, _N), jnp.float32),
        scratch_types=[
            pltpu.VMEM((_N,), jnp.float32),
            pltpu.VMEM((_N,), jnp.float32),
            pltpu.VMEM((_N + _L,), jnp.float32),
            pltpu.SemaphoreType.DMA,
            pltpu.SemaphoreType.DMA,
            pltpu.SemaphoreType.DMA,
            pltpu.SemaphoreType.DMA,
        ],
        compiler_params=pltpu.CompilerParams(needs_layout_passes=False),
    )
    def run(x_hbm, out_hbm, row_a, row_b, cand_v, si_a, si_b, so_a, so_b):
        wid = lax.axis_index("s") * 2 + lax.axis_index("c")
        base = wid * _ROWS_PER_W
        bufs = (row_a, row_b)
        sin = (si_a, si_b)
        sout = (so_a, so_b)

        def cp_in(r, b):
            return pltpu.make_async_copy(x_hbm.at[base + r], bufs[b], sin[b])

        def cp_out(r, b):
            return pltpu.make_async_copy(bufs[b], out_hbm.at[base + r], sout[b])

        cp_in(0, 0).start()
        cp_in(1, 1).start()
        for r in range(_ROWS_PER_W):
            b = r % 2
            cp_in(r, b).wait()
            if r >= 1 and r + 1 < _ROWS_PER_W:
                cp_out(r - 1, 1 - b).wait()
                cp_in(r + 1, 1 - b).start()
            _row_sparsemax(bufs[b], cand_v)
            cp_out(r, b).start()
        cp_out(_ROWS_PER_W - 2, _ROWS_PER_W % 2).wait()
        cp_out(_ROWS_PER_W - 1, 1 - _ROWS_PER_W % 2).wait()

    return run(inputs)
